# Initial kernel scaffold; baseline (speedup 1.0000x reference)
import functools

import jax
import jax.numpy as jnp
from jax import lax
from jax.experimental import pallas as pl
from jax.experimental.pallas import tpu as pltpu
from jax.experimental.pallas import tpu_sc as plsc

F = 26
V = 100000
D = 32
B = 4096
L = 20
NW = 32
BPW = B // NW
SB = 32
SUB = BPW // SB
ROWS = SB * L
KCH = ROWS // 128


def _build_kernel():
    mesh = plsc.VectorSubcoreMesh(core_axis_name="c", subcore_axis_name="s")

    @functools.partial(
        pl.kernel,
        out_type=jax.ShapeDtypeStruct((B, F * D), jnp.float32),
        mesh=mesh,
        scratch_types=[
            pltpu.VMEM((F, KCH, 128), jnp.int32),
            pltpu.VMEM((ROWS, D), jnp.float32),
            pltpu.VMEM((ROWS, D), jnp.float32),
            pltpu.VMEM((SB, F * D), jnp.float32),
            pltpu.SemaphoreType.DMA,
            pltpu.SemaphoreType.DMA,
        ],
    )
    def k(idx_hbm, tab_hbm, out_hbm, idx_v, rows0, rows1, acc_v, sem0, sem1):
        nc = 2
        wid = lax.axis_index("s") * nc + lax.axis_index("c")

        def issue(f, rbuf, sem):
            for kk in range(KCH):
                pltpu.async_copy(
                    tab_hbm.at[f].at[idx_v.at[f, kk]],
                    rbuf.at[pl.ds(kk * 128, 128)],
                    sem,
                )

        def drain(f, rbuf, sem):
            for kk in range(KCH):
                pltpu.make_async_copy(
                    tab_hbm.at[f].at[idx_v.at[f, kk]],
                    rbuf.at[pl.ds(kk * 128, 128)],
                    sem,
                ).wait()

        def pool(f, rbuf):
            def bag(j, _):
                base = j * L
                a0 = rbuf[base, pl.ds(0, 16)]
                a1 = rbuf[base, pl.ds(16, 16)]
                for l in range(1, L):
                    a0 = a0 + rbuf[base + l, pl.ds(0, 16)]
                    a1 = a1 + rbuf[base + l, pl.ds(16, 16)]
                acc_v[j, pl.ds(f * D, 16)] = a0
                acc_v[j, pl.ds(f * D + 16, 16)] = a1
                return 0

            lax.fori_loop(0, SB, bag, 0)

        def sub(s, _):
            pltpu.sync_copy(idx_hbm.at[:, wid, s], idx_v)
            issue(0, rows0, sem0)

            def pair(f2, _):
                fa = 2 * f2
                issue(fa + 1, rows1, sem1)
                drain(fa, rows0, sem0)
                pool(fa, rows0)
                issue(fa + 2, rows0, sem0)
                drain(fa + 1, rows1, sem1)
                pool(fa + 1, rows1)
                return 0

            lax.fori_loop(0, F // 2 - 1, pair, 0)
            issue(F - 1, rows1, sem1)
            drain(F - 2, rows0, sem0)
            pool(F - 2, rows0)
            drain(F - 1, rows1, sem1)
            pool(F - 1, rows1)

            gb = wid * BPW + s * SB
            pltpu.sync_copy(acc_v, out_hbm.at[pl.ds(gb, SB)])
            return 0

        lax.fori_loop(0, SUB, sub, 0)

    return k


_k = _build_kernel()


def kernel(indices, tables):
    idx5 = indices.astype(jnp.int32).reshape(F, NW, SUB, KCH, 128)
    return _k(idx5, tables)


# speedup vs baseline: 7.6269x; 7.6269x over previous
import functools

import jax
import jax.numpy as jnp
from jax import lax
from jax.experimental import pallas as pl
from jax.experimental.pallas import tpu as pltpu
from jax.experimental.pallas import tpu_sc as plsc

F = 26
V = 100000
D = 32
B = 4096
L = 20
NW = 32
BPW = B // NW
SB = 32
SUB = BPW // SB
ROWS = SB * L
KCH = ROWS // 128


def _build_kernel():
    mesh = plsc.VectorSubcoreMesh(core_axis_name="c", subcore_axis_name="s")

    @functools.partial(
        pl.kernel,
        out_type=jax.ShapeDtypeStruct((B, F * D), jnp.float32),
        mesh=mesh,
        compiler_params=pltpu.CompilerParams(use_tc_tiling_on_sc=False),
        scratch_types=[
            pltpu.VMEM((F, KCH, 128), jnp.int32),
            pltpu.VMEM((ROWS, D), jnp.float32),
            pltpu.VMEM((ROWS, D), jnp.float32),
            pltpu.VMEM((SB, F * D), jnp.float32),
            pltpu.SemaphoreType.DMA,
            pltpu.SemaphoreType.DMA,
        ],
    )
    def k(idx_hbm, tab_hbm, out_hbm, idx_v, rows0, rows1, acc_v, sem0, sem1):
        nc = 2
        wid = lax.axis_index("s") * nc + lax.axis_index("c")

        def issue(f, rbuf, sem):
            for kk in range(KCH):
                pltpu.async_copy(
                    tab_hbm.at[f].at[idx_v.at[f, kk]],
                    rbuf.at[pl.ds(kk * 128, 128)],
                    sem,
                )

        def drain(f, rbuf, sem):
            for kk in range(KCH):
                pltpu.make_async_copy(
                    tab_hbm.at[f].at[idx_v.at[f, kk]],
                    rbuf.at[pl.ds(kk * 128, 128)],
                    sem,
                ).wait()

        def pool(f, rbuf):
            def bag(j, _):
                base = j * L
                a0 = rbuf[base, pl.ds(0, 16)]
                a1 = rbuf[base, pl.ds(16, 16)]
                for l in range(1, L):
                    a0 = a0 + rbuf[base + l, pl.ds(0, 16)]
                    a1 = a1 + rbuf[base + l, pl.ds(16, 16)]
                acc_v[j, pl.ds(f * D, 16)] = a0
                acc_v[j, pl.ds(f * D + 16, 16)] = a1
                return 0

            lax.fori_loop(0, SB, bag, 0)

        def sub(s, _):
            pltpu.sync_copy(idx_hbm.at[:, wid, s], idx_v)
            issue(0, rows0, sem0)

            def pair(f2, _):
                fa = 2 * f2
                issue(fa + 1, rows1, sem1)
                drain(fa, rows0, sem0)
                pool(fa, rows0)
                issue(fa + 2, rows0, sem0)
                drain(fa + 1, rows1, sem1)
                pool(fa + 1, rows1)
                return 0

            lax.fori_loop(0, F // 2 - 1, pair, 0)
            issue(F - 1, rows1, sem1)
            drain(F - 2, rows0, sem0)
            pool(F - 2, rows0)
            drain(F - 1, rows1, sem1)
            pool(F - 1, rows1)

            gb = wid * BPW + s * SB
            pltpu.sync_copy(acc_v, out_hbm.at[pl.ds(gb, SB)])
            return 0

        lax.fori_loop(0, SUB, sub, 0)

    return k


_k = _build_kernel()


def kernel(indices, tables):
    idx5 = indices.astype(jnp.int32).reshape(F, NW, SUB, KCH, 128)
    return _k(idx5, tables)


# speedup vs baseline: 7.6310x; 1.0005x over previous
import functools

import jax
import jax.numpy as jnp
from jax import lax
from jax.experimental import pallas as pl
from jax.experimental.pallas import tpu as pltpu
from jax.experimental.pallas import tpu_sc as plsc

F = 26
V = 100000
D = 32
B = 4096
L = 20
NW = 32
BPW = B // NW
SB = 32
SUB = BPW // SB
ROWS = SB * L
KCH = ROWS // 128

def _build_gather():
    mesh = plsc.VectorSubcoreMesh(core_axis_name="c", subcore_axis_name="s")

    @functools.partial(
        pl.kernel,
        out_type=jax.ShapeDtypeStruct((B, F * D), jnp.float32),
        mesh=mesh,
        compiler_params=pltpu.CompilerParams(use_tc_tiling_on_sc=False),
        scratch_types=[
            pltpu.VMEM((F, KCH, 128), jnp.int32),
            pltpu.VMEM((ROWS, D), jnp.float32),
            pltpu.VMEM((ROWS, D), jnp.float32),
            pltpu.VMEM((SB, F * D), jnp.float32),
            pltpu.SemaphoreType.DMA,
            pltpu.SemaphoreType.DMA,
        ],
    )
    def k2(idx_hbm, tab_hbm, out_hbm, idx_v, rows0, rows1, acc_v, sem0, sem1):
        nc = 2
        wid = lax.axis_index("s") * nc + lax.axis_index("c")

        def issue(f, rbuf, sem):
            for kk in range(KCH):
                pltpu.async_copy(
                    tab_hbm.at[f].at[idx_v.at[f, kk]],
                    rbuf.at[pl.ds(kk * 128, 128)],
                    sem,
                )

        def drain(f, rbuf, sem):
            for kk in range(KCH):
                pltpu.make_async_copy(
                    tab_hbm.at[f].at[idx_v.at[f, kk]],
                    rbuf.at[pl.ds(kk * 128, 128)],
                    sem,
                ).wait()

        def pool(f, rbuf):
            def bag(j, _):
                base = j * L
                a0 = rbuf[base, pl.ds(0, 16)]
                a1 = rbuf[base, pl.ds(16, 16)]
                for l in range(1, L):
                    a0 = a0 + rbuf[base + l, pl.ds(0, 16)]
                    a1 = a1 + rbuf[base + l, pl.ds(16, 16)]
                acc_v[j, pl.ds(f * D, 16)] = a0
                acc_v[j, pl.ds(f * D + 16, 16)] = a1
                return 0

            lax.fori_loop(0, SB, bag, 0)

        def sub(s, _):
            pltpu.sync_copy(idx_hbm.at[:, wid, s], idx_v)
            issue(0, rows0, sem0)

            def pair(f2, _):
                fa = 2 * f2
                issue(fa + 1, rows1, sem1)
                drain(fa, rows0, sem0)
                pool(fa, rows0)
                issue(fa + 2, rows0, sem0)
                drain(fa + 1, rows1, sem1)
                pool(fa + 1, rows1)
                return 0

            lax.fori_loop(0, F // 2 - 1, pair, 0)
            issue(F - 1, rows1, sem1)
            drain(F - 2, rows0, sem0)
            pool(F - 2, rows0)
            drain(F - 1, rows1, sem1)
            pool(F - 1, rows1)

            gb = wid * BPW + s * SB
            pltpu.sync_copy(acc_v, out_hbm.at[pl.ds(gb, SB)])
            return 0

        lax.fori_loop(0, SUB, sub, 0)

    return k2


_k2 = _build_gather()


def kernel(indices, tables):
    idx5 = indices.astype(jnp.int32).reshape(F, NW, SUB, KCH, 128)
    return _k2(idx5, tables)


# speedup vs baseline: 7.7664x; 1.0177x over previous
import functools

import jax
import jax.numpy as jnp
from jax import lax
from jax.experimental import pallas as pl
from jax.experimental.pallas import tpu as pltpu
from jax.experimental.pallas import tpu_sc as plsc

F = 26
V = 100000
D = 32
B = 4096
L = 20
NW = 32
BPW = B // NW
SB = 32
SUB = BPW // SB
ROWS = SB * L
KCH = ROWS // 128

def _build_gather():
    mesh = plsc.VectorSubcoreMesh(core_axis_name="c", subcore_axis_name="s")

    @functools.partial(
        pl.kernel,
        out_type=jax.ShapeDtypeStruct((B, F * D), jnp.float32),
        mesh=mesh,
        compiler_params=pltpu.CompilerParams(use_tc_tiling_on_sc=False),
        scratch_types=[
            pltpu.VMEM((F, L, SB), jnp.int32),
            pltpu.VMEM((F, KCH, 128), jnp.int32),
            pltpu.VMEM((ROWS, D), jnp.float32),
            pltpu.VMEM((ROWS, D), jnp.float32),
            pltpu.VMEM((SB, F * D), jnp.float32),
            pltpu.SemaphoreType.DMA,
            pltpu.SemaphoreType.DMA,
        ],
    )
    def k2(idx_hbm, tab_hbm, out_hbm, idx_v, idx_ck, rows0, rows1, acc_v,
           sem0, sem1):
        nc = 2
        wid = lax.axis_index("s") * nc + lax.axis_index("c")

        def repack(f, _):
            for l in range(L):
                for h in (0, 16):
                    flat = l * SB + h
                    idx_ck[f, flat // 128, pl.ds(flat % 128, 16)] = (
                        idx_v[f, l, pl.ds(h, 16)])
            return 0

        def issue(f, rbuf, sem):
            for kk in range(KCH):
                pltpu.async_copy(
                    tab_hbm.at[f].at[idx_ck.at[f, kk]],
                    rbuf.at[pl.ds(kk * 128, 128)],
                    sem,
                )

        def drain(f, rbuf, sem):
            for kk in range(KCH):
                pltpu.make_async_copy(
                    tab_hbm.at[f].at[idx_ck.at[f, kk]],
                    rbuf.at[pl.ds(kk * 128, 128)],
                    sem,
                ).wait()

        def pool(f, rbuf):
            def bag(j, _):
                a0 = rbuf[j, pl.ds(0, 16)]
                a1 = rbuf[j, pl.ds(16, 16)]
                for l in range(1, L):
                    a0 = a0 + rbuf[l * SB + j, pl.ds(0, 16)]
                    a1 = a1 + rbuf[l * SB + j, pl.ds(16, 16)]
                acc_v[j, pl.ds(f * D, 16)] = a0
                acc_v[j, pl.ds(f * D + 16, 16)] = a1
                return 0

            lax.fori_loop(0, SB, bag, 0)

        def sub(s, _):
            gb = wid * BPW + s * SB
            pltpu.sync_copy(idx_hbm.at[:, :, pl.ds(gb, SB)], idx_v)
            lax.fori_loop(0, F, repack, 0)
            issue(0, rows0, sem0)

            def pair(f2, _):
                fa = 2 * f2
                issue(fa + 1, rows1, sem1)
                drain(fa, rows0, sem0)
                pool(fa, rows0)
                issue(fa + 2, rows0, sem0)
                drain(fa + 1, rows1, sem1)
                pool(fa + 1, rows1)
                return 0

            lax.fori_loop(0, F // 2 - 1, pair, 0)
            issue(F - 1, rows1, sem1)
            drain(F - 2, rows0, sem0)
            pool(F - 2, rows0)
            drain(F - 1, rows1, sem1)
            pool(F - 1, rows1)

            pltpu.sync_copy(acc_v, out_hbm.at[pl.ds(gb, SB)])
            return 0

        lax.fori_loop(0, SUB, sub, 0)

    return k2


_k2 = _build_gather()


def kernel(indices, tables):
    idxT = jnp.transpose(indices.astype(jnp.int32), (0, 2, 1))
    return _k2(idxT, tables)
